# baseline (speedup 1.0000x reference)
import functools

import jax
import jax.numpy as jnp
from jax import lax
from jax.experimental import pallas as pl
from jax.experimental.pallas import tpu as pltpu
from jax.experimental.pallas import tpu_sc as plsc

B = 64
V = 1_000_000
EPSILON = 0.05

NW = 32
ROWS_PER_W = B // NW
CHUNK = 40_000
NCHUNK = V // CHUNK
NVREG = CHUNK // 16

_mesh = plsc.VectorSubcoreMesh(core_axis_name="c", subcore_axis_name="s")

_GATHER_DNUMS = lax.GatherDimensionNumbers(
    offset_dims=(), collapsed_slice_dims=(0,), start_index_map=(0,))


def _perm16(x, pidx):
    return lax.gather(x, pidx[:, None], _GATHER_DNUMS, (1,),
                      mode=lax.GatherScatterMode.PROMISE_IN_BOUNDS)


@functools.partial(
    pl.kernel,
    mesh=_mesh,
    out_type=jax.ShapeDtypeStruct((NW, 16), jnp.int32),
    scratch_types=[
        pltpu.VMEM((CHUNK,), jnp.float32),
        pltpu.VMEM((16,), jnp.int32),
    ],
)
def _sc_argmax(x_hbm, out_hbm, buf, res):
    wid = lax.axis_index("c") * 16 + lax.axis_index("s")
    lane = lax.iota(jnp.int32, 16)

    bests = []
    for r in range(ROWS_PER_W):
        row = wid * ROWS_PER_W + r
        row_base = row * V

        def chunk_body(c, carry):
            m, idx, pos = carry
            pltpu.sync_copy(x_hbm.at[pl.ds(row_base + c * CHUNK, CHUNK)], buf)

            def vreg_body(j, carry):
                m, idx, pos = carry
                v = buf[pl.ds(j * 16, 16)]
                gt = v > m
                m = jnp.where(gt, v, m)
                idx = jnp.where(gt, pos, idx)
                return m, idx, pos + 16

            return lax.fori_loop(0, NVREG, vreg_body, (m, idx, pos))

        m0 = jnp.full((16,), -jnp.inf, jnp.float32)
        i0 = jnp.zeros((16,), jnp.int32)
        m, idx, _ = lax.fori_loop(0, NCHUNK, chunk_body, (m0, i0, lane))

        for d in (1, 2, 4, 8):
            pidx = lane ^ d
            mp = _perm16(m, pidx)
            ip = _perm16(idx, pidx)
            better = (mp > m) | ((mp == m) & (ip < idx))
            m = jnp.where(better, mp, m)
            idx = jnp.where(better, ip, idx)
        bests.append(idx)

    vec = jnp.where(lane == 0, bests[0], jnp.where(lane == 1, bests[1], 0))
    res[...] = vec
    pltpu.sync_copy(res, out_hbm.at[wid])


def kernel(x):
    staged = _sc_argmax(x.reshape(B * V))
    bests = staged[:, :ROWS_PER_W].reshape(B, 1)
    k1 = jax.random.key(1)
    k2 = jax.random.key(2)
    sampled = jax.random.randint(k1, (B,), 0, V, dtype=jnp.int32)
    bern = jax.random.bernoulli(k2, 1.0 - EPSILON, (B, 1)).astype(jnp.int32)
    return bests * bern + (1 - bern) * sampled


# speedup vs baseline: 32.9865x; 32.9865x over previous
import functools

import jax
import jax.numpy as jnp
from jax import lax
from jax.experimental import pallas as pl
from jax.experimental.pallas import tpu as pltpu
from jax.experimental.pallas import tpu_sc as plsc

B = 64
V = 1_000_000
EPSILON = 0.05

NW = 32
LANE = 128
FULL_TILES = V // LANE
TAIL = V - FULL_TILES * LANE
QT = FULL_TILES // 4
QCOLS = QT * LANE
CT = 31
CHUNKC = CT * LANE
NCH = QT // CT
JGROUPS = CHUNKC // 64

_mesh = plsc.VectorSubcoreMesh(core_axis_name="c", subcore_axis_name="s")

_GATHER_DNUMS = lax.GatherDimensionNumbers(
    offset_dims=(), collapsed_slice_dims=(0,), start_index_map=(0,))


def _perm16(x, pidx):
    return lax.gather(x, pidx[:, None], _GATHER_DNUMS, (1,),
                      mode=lax.GatherScatterMode.PROMISE_IN_BOUNDS)


@functools.partial(
    pl.kernel,
    mesh=_mesh,
    out_type=[
        jax.ShapeDtypeStruct((NW, 16), jnp.float32),
        jax.ShapeDtypeStruct((NW, 16), jnp.int32),
    ],
    scratch_types=[
        pltpu.VMEM((8, CHUNKC), jnp.float32),
        pltpu.VMEM((8, CHUNKC), jnp.float32),
        pltpu.VMEM((8, TAIL), jnp.float32),
        pltpu.VMEM((16,), jnp.float32),
        pltpu.VMEM((16,), jnp.int32),
        pltpu.SemaphoreType.DMA,
        pltpu.SemaphoreType.DMA,
    ],
)
def _sc_argmax(x_hbm, maxs_hbm, idxs_hbm, buf0, buf1, tailbuf, resm, resi,
               sem0, sem1):
    c = lax.axis_index("c")
    s = lax.axis_index("s")
    wid = c * 16 + s
    rb = c * 4 + s // 4
    kq = s % 4
    row0 = rb * 8
    lane = lax.iota(jnp.int32, 16)

    def start_dma(t, b, sem):
        col0 = kq * QCOLS + t * CHUNKC
        pltpu.make_async_copy(
            x_hbm.at[pl.ds(row0, 8), pl.ds(col0, CHUNKC)], b, sem).start()

    def wait_dma(b, sem):
        pltpu.make_async_copy(
            x_hbm.at[pl.ds(row0, 8), pl.ds(0, CHUNKC)], b, sem).wait()

    def process(b, t, ms, idxs):
        col0 = kq * QCOLS + t * CHUNKC
        pos0 = col0 + lane

        def jbody(j, carry):
            ms, idxs, pos = carry
            ms = list(ms)
            idxs = list(idxs)
            base = j * 64
            for du in range(4):
                p = pos + (du * 16) if du else pos
                for r in range(8):
                    v = b[r, pl.ds(base + du * 16, 16)]
                    gt = v > ms[r]
                    ms[r] = jnp.where(gt, v, ms[r])
                    idxs[r] = jnp.where(gt, p, idxs[r])
            return tuple(ms), tuple(idxs), pos + 64

        ms, idxs, _ = lax.fori_loop(0, JGROUPS, jbody, (ms, idxs, pos0))
        return ms, idxs

    ms = tuple(jnp.full((16,), -jnp.inf, jnp.float32) for _ in range(8))
    idxs = tuple(jnp.zeros((16,), jnp.int32) for _ in range(8))

    start_dma(0, buf0, sem0)

    def ubody(u, carry):
        ms, idxs = carry
        t0 = u * 2
        start_dma(t0 + 1, buf1, sem1)
        wait_dma(buf0, sem0)
        ms, idxs = process(buf0, t0, ms, idxs)

        @pl.when(t0 + 2 < NCH)
        def _():
            start_dma(t0 + 2, buf0, sem0)

        wait_dma(buf1, sem1)
        ms, idxs = process(buf1, t0 + 1, ms, idxs)
        return ms, idxs

    ms, idxs = lax.fori_loop(0, NCH // 2, ubody, (ms, idxs))
    wait_dma(buf0, sem0)
    ms, idxs = process(buf0, NCH - 1, ms, idxs)

    pltpu.sync_copy(x_hbm.at[pl.ds(row0, 8), pl.ds(FULL_TILES * LANE, TAIL)],
                    tailbuf)
    penalty = jnp.where(kq == 3, jnp.float32(0), jnp.float32(-jnp.inf))
    ms = list(ms)
    idxs = list(idxs)
    for j in range(TAIL // 16):
        p = FULL_TILES * LANE + j * 16 + lane
        for r in range(8):
            v = tailbuf[r, pl.ds(j * 16, 16)] + penalty
            gt = v > ms[r]
            ms[r] = jnp.where(gt, v, ms[r])
            idxs[r] = jnp.where(gt, p, idxs[r])

    for r in range(8):
        m, idx = ms[r], idxs[r]
        for d in (1, 2, 4, 8):
            pidx = lane ^ d
            mp = _perm16(m, pidx)
            ip = _perm16(idx, pidx)
            better = (mp > m) | ((mp == m) & (ip < idx))
            m = jnp.where(better, mp, m)
            idx = jnp.where(better, ip, idx)
        ms[r], idxs[r] = m, idx

    accm = jnp.full((16,), -jnp.inf, jnp.float32)
    acci = jnp.zeros((16,), jnp.int32)
    for r in range(8):
        accm = jnp.where(lane == r, ms[r], accm)
        acci = jnp.where(lane == r, idxs[r], acci)
    resm[...] = accm
    resi[...] = acci
    pltpu.sync_copy(resm, maxs_hbm.at[wid])
    pltpu.sync_copy(resi, idxs_hbm.at[wid])


def kernel(x):
    staged_m, staged_i = _sc_argmax(x)
    m4 = staged_m[:, :8].reshape(2, 4, 4, 8).transpose(0, 1, 3, 2).reshape(B, 4)
    i4 = staged_i[:, :8].reshape(2, 4, 4, 8).transpose(0, 1, 3, 2).reshape(B, 4)
    rowmax = m4.max(axis=1, keepdims=True)
    cand = jnp.where(m4 == rowmax, i4, jnp.int32(2**31 - 1))
    bests = cand.min(axis=1).reshape(B, 1)

    k1 = jax.random.key(1)
    k2 = jax.random.key(2)
    sampled = jax.random.randint(k1, (B,), 0, V, dtype=jnp.int32)
    bern = jax.random.bernoulli(k2, 1.0 - EPSILON, (B, 1)).astype(jnp.int32)
    return bests * bern + (1 - bern) * sampled


# speedup vs baseline: 36.4195x; 1.1041x over previous
import functools

import jax
import jax.numpy as jnp
from jax import lax
from jax.experimental import pallas as pl
from jax.experimental.pallas import tpu as pltpu
from jax.experimental.pallas import tpu_sc as plsc

B = 64
V = 1_000_000
EPSILON = 0.05

NW = 32
LANE = 128
FULL_TILES = V // LANE
TAIL = V - FULL_TILES * LANE
QT = FULL_TILES // 4
QCOLS = QT * LANE
CT = 31
CHUNKC = CT * LANE
NCH = QT // CT
JGROUPS = CHUNKC // 64

_mesh = plsc.VectorSubcoreMesh(core_axis_name="c", subcore_axis_name="s")

_GATHER_DNUMS = lax.GatherDimensionNumbers(
    offset_dims=(), collapsed_slice_dims=(0,), start_index_map=(0,))


def _perm16(x, pidx):
    return lax.gather(x, pidx[:, None], _GATHER_DNUMS, (1,),
                      mode=lax.GatherScatterMode.PROMISE_IN_BOUNDS)


@functools.partial(
    pl.kernel,
    mesh=_mesh,
    out_type=[
        jax.ShapeDtypeStruct((NW, 16), jnp.float32),
        jax.ShapeDtypeStruct((NW, 16), jnp.int32),
    ],
    scratch_types=[
        pltpu.VMEM((8, CHUNKC), jnp.float32),
        pltpu.VMEM((8, CHUNKC), jnp.float32),
        pltpu.VMEM((8, CHUNKC), jnp.float32),
        pltpu.VMEM((8, TAIL), jnp.float32),
        pltpu.VMEM((16,), jnp.float32),
        pltpu.VMEM((16,), jnp.int32),
        pltpu.SemaphoreType.DMA,
        pltpu.SemaphoreType.DMA,
        pltpu.SemaphoreType.DMA,
    ],
)
def _sc_argmax(x_hbm, maxs_hbm, idxs_hbm, buf0, buf1, buf2, tailbuf,
               resm, resi, sem0, sem1, sem2):
    c = lax.axis_index("c")
    s = lax.axis_index("s")
    wid = c * 16 + s
    rb = c * 4 + s // 4
    kq = s % 4
    row0 = rb * 8
    lane = lax.iota(jnp.int32, 16)

    def start_dma(t, b, sem):
        col0 = kq * QCOLS + t * CHUNKC
        pltpu.make_async_copy(
            x_hbm.at[pl.ds(row0, 8), pl.ds(col0, CHUNKC)], b, sem).start()

    def wait_dma(b, sem):
        pltpu.make_async_copy(
            x_hbm.at[pl.ds(row0, 8), pl.ds(0, CHUNKC)], b, sem).wait()

    def process(b, t, ms, idxs):
        col0 = kq * QCOLS + t * CHUNKC
        pos0 = col0 + lane

        def jbody(j, carry):
            ms, idxs, pos = carry
            ms = list(ms)
            idxs = list(idxs)
            base = j * 64
            for du in range(4):
                p = pos + (du * 16) if du else pos
                for r in range(8):
                    v = b[r, pl.ds(base + du * 16, 16)]
                    gt = v > ms[r]
                    ms[r] = jnp.where(gt, v, ms[r])
                    idxs[r] = jnp.where(gt, p, idxs[r])
            return tuple(ms), tuple(idxs), pos + 64

        ms, idxs, _ = lax.fori_loop(0, JGROUPS, jbody, (ms, idxs, pos0))
        return ms, idxs

    ms = tuple(jnp.full((16,), -jnp.inf, jnp.float32) for _ in range(8))
    idxs = tuple(jnp.zeros((16,), jnp.int32) for _ in range(8))

    bufs = (buf0, buf1, buf2)
    sems = (sem0, sem1, sem2)
    start_dma(0, buf0, sem0)
    start_dma(1, buf1, sem1)

    def ubody(u, carry):
        ms, idxs = carry
        t0 = u * 3
        for q in range(3):
            nxt = t0 + 2 + q

            @pl.when(nxt < NCH)
            def _(nxt=nxt, q=q):
                start_dma(nxt, bufs[(2 + q) % 3], sems[(2 + q) % 3])

            wait_dma(bufs[q], sems[q])
            ms, idxs = process(bufs[q], t0 + q, ms, idxs)
        return ms, idxs

    ms, idxs = lax.fori_loop(0, NCH // 3, ubody, (ms, idxs))

    pltpu.sync_copy(x_hbm.at[pl.ds(row0, 8), pl.ds(FULL_TILES * LANE, TAIL)],
                    tailbuf)
    penalty = jnp.where(kq == 3, jnp.float32(0), jnp.float32(-jnp.inf))
    ms = list(ms)
    idxs = list(idxs)
    for j in range(TAIL // 16):
        p = FULL_TILES * LANE + j * 16 + lane
        for r in range(8):
            v = tailbuf[r, pl.ds(j * 16, 16)] + penalty
            gt = v > ms[r]
            ms[r] = jnp.where(gt, v, ms[r])
            idxs[r] = jnp.where(gt, p, idxs[r])

    for r in range(8):
        m, idx = ms[r], idxs[r]
        for d in (1, 2, 4, 8):
            pidx = lane ^ d
            mp = _perm16(m, pidx)
            ip = _perm16(idx, pidx)
            better = (mp > m) | ((mp == m) & (ip < idx))
            m = jnp.where(better, mp, m)
            idx = jnp.where(better, ip, idx)
        ms[r], idxs[r] = m, idx

    accm = jnp.full((16,), -jnp.inf, jnp.float32)
    acci = jnp.zeros((16,), jnp.int32)
    for r in range(8):
        accm = jnp.where(lane == r, ms[r], accm)
        acci = jnp.where(lane == r, idxs[r], acci)
    resm[...] = accm
    resi[...] = acci
    pltpu.sync_copy(resm, maxs_hbm.at[wid])
    pltpu.sync_copy(resi, idxs_hbm.at[wid])


def kernel(x):
    staged_m, staged_i = _sc_argmax(x)
    m4 = staged_m[:, :8].reshape(2, 4, 4, 8).transpose(0, 1, 3, 2).reshape(B, 4)
    i4 = staged_i[:, :8].reshape(2, 4, 4, 8).transpose(0, 1, 3, 2).reshape(B, 4)
    rowmax = m4.max(axis=1, keepdims=True)
    cand = jnp.where(m4 == rowmax, i4, jnp.int32(2**31 - 1))
    bests = cand.min(axis=1).reshape(B, 1)

    k1 = jax.random.key(1)
    k2 = jax.random.key(2)
    sampled = jax.random.randint(k1, (B,), 0, V, dtype=jnp.int32)
    bern = jax.random.bernoulli(k2, 1.0 - EPSILON, (B, 1)).astype(jnp.int32)
    return bests * bern + (1 - bern) * sampled


# speedup vs baseline: 40.3239x; 1.1072x over previous
import functools

import jax
import jax.numpy as jnp
from jax import lax
from jax.experimental import pallas as pl
from jax.experimental.pallas import tpu as pltpu
from jax.experimental.pallas import tpu_sc as plsc

B = 64
V = 1_000_000
EPSILON = 0.05

NW = 32
LANE = 128
FULL_TILES = V // LANE
TAIL = V - FULL_TILES * LANE
CT = 31
CHUNKC = CT * LANE
NCH = 33
QT = NCH * CT
QCOLS = QT * LANE
CSC = 4 * QCOLS
JGROUPS = CHUNKC // 64

TC_BLK = CHUNKC
TC_BLK0 = CSC // TC_BLK
TC_STEPS = (FULL_TILES * LANE - CSC) // TC_BLK

_mesh = plsc.VectorSubcoreMesh(core_axis_name="c", subcore_axis_name="s")

_GATHER_DNUMS = lax.GatherDimensionNumbers(
    offset_dims=(), collapsed_slice_dims=(0,), start_index_map=(0,))


def _perm16(x, pidx):
    return lax.gather(x, pidx[:, None], _GATHER_DNUMS, (1,),
                      mode=lax.GatherScatterMode.PROMISE_IN_BOUNDS)


@functools.partial(
    pl.kernel,
    mesh=_mesh,
    out_type=[
        jax.ShapeDtypeStruct((NW, 16), jnp.float32),
        jax.ShapeDtypeStruct((NW, 16), jnp.int32),
    ],
    scratch_types=[
        pltpu.VMEM((8, CHUNKC), jnp.float32),
        pltpu.VMEM((8, CHUNKC), jnp.float32),
        pltpu.VMEM((8, CHUNKC), jnp.float32),
        pltpu.VMEM((8, TAIL), jnp.float32),
        pltpu.VMEM((16,), jnp.float32),
        pltpu.VMEM((16,), jnp.int32),
        pltpu.SemaphoreType.DMA,
        pltpu.SemaphoreType.DMA,
        pltpu.SemaphoreType.DMA,
    ],
)
def _sc_argmax(x_hbm, maxs_hbm, idxs_hbm, buf0, buf1, buf2, tailbuf,
               resm, resi, sem0, sem1, sem2):
    c = lax.axis_index("c")
    s = lax.axis_index("s")
    wid = c * 16 + s
    rb = c * 4 + s // 4
    kq = s % 4
    row0 = rb * 8
    lane = lax.iota(jnp.int32, 16)

    def start_dma(t, b, sem):
        col0 = kq * QCOLS + t * CHUNKC
        pltpu.make_async_copy(
            x_hbm.at[pl.ds(row0, 8), pl.ds(col0, CHUNKC)], b, sem).start()

    def wait_dma(b, sem):
        pltpu.make_async_copy(
            x_hbm.at[pl.ds(row0, 8), pl.ds(0, CHUNKC)], b, sem).wait()

    def process(b, t, ms, idxs):
        col0 = kq * QCOLS + t * CHUNKC
        pos0 = col0 + lane

        def jbody(j, carry):
            ms, idxs, pos = carry
            ms = list(ms)
            idxs = list(idxs)
            base = j * 64
            for du in range(4):
                p = pos + (du * 16) if du else pos
                for r in range(8):
                    v = b[r, pl.ds(base + du * 16, 16)]
                    gt = v > ms[r]
                    ms[r] = jnp.where(gt, v, ms[r])
                    idxs[r] = jnp.where(gt, p, idxs[r])
            return tuple(ms), tuple(idxs), pos + 64

        ms, idxs, _ = lax.fori_loop(0, JGROUPS, jbody, (ms, idxs, pos0))
        return ms, idxs

    ms = tuple(jnp.full((16,), -jnp.inf, jnp.float32) for _ in range(8))
    idxs = tuple(jnp.zeros((16,), jnp.int32) for _ in range(8))

    bufs = (buf0, buf1, buf2)
    sems = (sem0, sem1, sem2)
    start_dma(0, buf0, sem0)
    start_dma(1, buf1, sem1)

    def ubody(u, carry):
        ms, idxs = carry
        t0 = u * 3
        for q in range(3):
            nxt = t0 + 2 + q

            @pl.when(nxt < NCH)
            def _(nxt=nxt, q=q):
                start_dma(nxt, bufs[(2 + q) % 3], sems[(2 + q) % 3])

            wait_dma(bufs[q], sems[q])
            ms, idxs = process(bufs[q], t0 + q, ms, idxs)
        return ms, idxs

    ms, idxs = lax.fori_loop(0, NCH // 3, ubody, (ms, idxs))

    pltpu.sync_copy(x_hbm.at[pl.ds(row0, 8), pl.ds(FULL_TILES * LANE, TAIL)],
                    tailbuf)
    penalty = jnp.where(kq == 3, jnp.float32(0), jnp.float32(-jnp.inf))
    ms = list(ms)
    idxs = list(idxs)
    for j in range(TAIL // 16):
        p = FULL_TILES * LANE + j * 16 + lane
        for r in range(8):
            v = tailbuf[r, pl.ds(j * 16, 16)] + penalty
            gt = v > ms[r]
            ms[r] = jnp.where(gt, v, ms[r])
            idxs[r] = jnp.where(gt, p, idxs[r])

    for r in range(8):
        m, idx = ms[r], idxs[r]
        for d in (1, 2, 4, 8):
            pidx = lane ^ d
            mp = _perm16(m, pidx)
            ip = _perm16(idx, pidx)
            better = (mp > m) | ((mp == m) & (ip < idx))
            m = jnp.where(better, mp, m)
            idx = jnp.where(better, ip, idx)
        ms[r], idxs[r] = m, idx

    accm = jnp.full((16,), -jnp.inf, jnp.float32)
    acci = jnp.zeros((16,), jnp.int32)
    for r in range(8):
        accm = jnp.where(lane == r, ms[r], accm)
        acci = jnp.where(lane == r, idxs[r], acci)
    resm[...] = accm
    resi[...] = acci
    pltpu.sync_copy(resm, maxs_hbm.at[wid])
    pltpu.sync_copy(resi, idxs_hbm.at[wid])


def _tc_body(x_ref, maxs_ref, idxs_ref, mstate, istate):
    i = pl.program_id(0)

    @pl.when(i == 0)
    def _():
        mstate[...] = jnp.full((B, 1), -jnp.inf, jnp.float32)
        istate[...] = jnp.zeros((B, 1), jnp.int32)

    blk = x_ref[...]
    bm = jnp.max(blk, axis=1, keepdims=True)
    cols = lax.broadcasted_iota(jnp.int32, (B, TC_BLK), 1)
    bi = jnp.min(jnp.where(blk == bm, cols, jnp.int32(2**31 - 1)),
                 axis=1, keepdims=True)
    m = mstate[...]
    gt = bm > m
    mstate[...] = jnp.where(gt, bm, m)
    istate[...] = jnp.where(gt, CSC + i * TC_BLK + bi, istate[...])

    @pl.when(i == TC_STEPS - 1)
    def _():
        maxs_ref[...] = mstate[...]
        idxs_ref[...] = istate[...]


_tc_argmax = pl.pallas_call(
    _tc_body,
    grid=(TC_STEPS,),
    in_specs=[pl.BlockSpec((B, TC_BLK), lambda i: (0, TC_BLK0 + i))],
    out_specs=[pl.BlockSpec((B, 1), lambda i: (0, 0)),
               pl.BlockSpec((B, 1), lambda i: (0, 0))],
    out_shape=[jax.ShapeDtypeStruct((B, 1), jnp.float32),
               jax.ShapeDtypeStruct((B, 1), jnp.int32)],
    scratch_shapes=[pltpu.VMEM((B, 1), jnp.float32),
                    pltpu.VMEM((B, 1), jnp.int32)],
)


def kernel(x):
    staged_m, staged_i = _sc_argmax(x)
    tc_m, tc_i = _tc_argmax(x)
    m4 = staged_m[:, :8].reshape(2, 4, 4, 8).transpose(0, 1, 3, 2).reshape(B, 4)
    i4 = staged_i[:, :8].reshape(2, 4, 4, 8).transpose(0, 1, 3, 2).reshape(B, 4)
    m5 = jnp.concatenate([m4, tc_m], axis=1)
    i5 = jnp.concatenate([i4, tc_i], axis=1)
    rowmax = m5.max(axis=1, keepdims=True)
    cand = jnp.where(m5 == rowmax, i5, jnp.int32(2**31 - 1))
    bests = cand.min(axis=1).reshape(B, 1)

    k1 = jax.random.key(1)
    k2 = jax.random.key(2)
    sampled = jax.random.randint(k1, (B,), 0, V, dtype=jnp.int32)
    bern = jax.random.bernoulli(k2, 1.0 - EPSILON, (B, 1)).astype(jnp.int32)
    return bests * bern + (1 - bern) * sampled


# speedup vs baseline: 43.0990x; 1.0688x over previous
import functools

import jax
import jax.numpy as jnp
from jax import lax
from jax.experimental import pallas as pl
from jax.experimental.pallas import tpu as pltpu
from jax.experimental.pallas import tpu_sc as plsc

B = 64
V = 1_000_000
EPSILON = 0.05

NW = 32
LANE = 128
FULL_TILES = V // LANE
TAIL = V - FULL_TILES * LANE
CT = 31
CHUNKC = CT * LANE
NCH = 33
QT = NCH * CT
QCOLS = QT * LANE
CSC = 4 * QCOLS
JGROUPS = CHUNKC // 64

TC_BLK = CHUNKC
TC_BLK0 = CSC // TC_BLK
TC_STEPS = (FULL_TILES * LANE - CSC) // TC_BLK

_mesh = plsc.VectorSubcoreMesh(core_axis_name="c", subcore_axis_name="s")

_GATHER_DNUMS = lax.GatherDimensionNumbers(
    offset_dims=(), collapsed_slice_dims=(0,), start_index_map=(0,))


def _perm16(x, pidx):
    return lax.gather(x, pidx[:, None], _GATHER_DNUMS, (1,),
                      mode=lax.GatherScatterMode.PROMISE_IN_BOUNDS)


@functools.partial(
    pl.kernel,
    mesh=_mesh,
    out_type=[
        jax.ShapeDtypeStruct((NW, 16), jnp.float32),
        jax.ShapeDtypeStruct((NW, 16), jnp.int32),
    ],
    scratch_types=[
        pltpu.VMEM((8, CHUNKC), jnp.float32),
        pltpu.VMEM((8, CHUNKC), jnp.float32),
        pltpu.VMEM((8, CHUNKC), jnp.float32),
        pltpu.VMEM((8, TAIL), jnp.float32),
        pltpu.VMEM((16,), jnp.float32),
        pltpu.VMEM((16,), jnp.int32),
        pltpu.SemaphoreType.DMA,
        pltpu.SemaphoreType.DMA,
        pltpu.SemaphoreType.DMA,
    ],
)
def _sc_argmax(x_hbm, maxs_hbm, idxs_hbm, buf0, buf1, buf2, tailbuf,
               resm, resi, sem0, sem1, sem2):
    c = lax.axis_index("c")
    s = lax.axis_index("s")
    wid = c * 16 + s
    rb = c * 4 + s // 4
    kq = s % 4
    row0 = rb * 8
    lane = lax.iota(jnp.int32, 16)

    def start_dma(t, b, sem):
        col0 = kq * QCOLS + t * CHUNKC
        pltpu.make_async_copy(
            x_hbm.at[pl.ds(row0, 8), pl.ds(col0, CHUNKC)], b, sem).start()

    def wait_dma(b, sem):
        pltpu.make_async_copy(
            x_hbm.at[pl.ds(row0, 8), pl.ds(0, CHUNKC)], b, sem).wait()

    def process(b, t, ms, idxs):
        col0 = kq * QCOLS + t * CHUNKC
        pos0 = col0 + lane

        def jbody(j, carry):
            ms, idxs, pos = carry
            ms = list(ms)
            idxs = list(idxs)
            base = j * 64
            for du in range(4):
                p = pos + (du * 16) if du else pos
                for r in range(8):
                    v = b[r, pl.ds(base + du * 16, 16)]
                    gt = v > ms[r]
                    ms[r] = jnp.where(gt, v, ms[r])
                    idxs[r] = jnp.where(gt, p, idxs[r])
            return tuple(ms), tuple(idxs), pos + 64

        ms, idxs, _ = lax.fori_loop(0, JGROUPS, jbody, (ms, idxs, pos0))
        return ms, idxs

    ms = tuple(jnp.full((16,), -jnp.inf, jnp.float32) for _ in range(8))
    idxs = tuple(jnp.zeros((16,), jnp.int32) for _ in range(8))

    bufs = (buf0, buf1, buf2)
    sems = (sem0, sem1, sem2)
    start_dma(0, buf0, sem0)
    start_dma(1, buf1, sem1)

    def ubody(u, carry):
        ms, idxs = carry
        t0 = u * 3
        for q in range(3):
            nxt = t0 + 2 + q

            @pl.when(nxt < NCH)
            def _(nxt=nxt, q=q):
                start_dma(nxt, bufs[(2 + q) % 3], sems[(2 + q) % 3])

            wait_dma(bufs[q], sems[q])
            ms, idxs = process(bufs[q], t0 + q, ms, idxs)
        return ms, idxs

    ms, idxs = lax.fori_loop(0, NCH // 3, ubody, (ms, idxs))

    pltpu.sync_copy(x_hbm.at[pl.ds(row0, 8), pl.ds(FULL_TILES * LANE, TAIL)],
                    tailbuf)
    penalty = jnp.where(kq == 3, jnp.float32(0), jnp.float32(-jnp.inf))
    ms = list(ms)
    idxs = list(idxs)
    for j in range(TAIL // 16):
        p = FULL_TILES * LANE + j * 16 + lane
        for r in range(8):
            v = tailbuf[r, pl.ds(j * 16, 16)] + penalty
            gt = v > ms[r]
            ms[r] = jnp.where(gt, v, ms[r])
            idxs[r] = jnp.where(gt, p, idxs[r])

    for r in range(8):
        m, idx = ms[r], idxs[r]
        for d in (1, 2, 4, 8):
            pidx = lane ^ d
            mp = _perm16(m, pidx)
            ip = _perm16(idx, pidx)
            better = (mp > m) | ((mp == m) & (ip < idx))
            m = jnp.where(better, mp, m)
            idx = jnp.where(better, ip, idx)
        ms[r], idxs[r] = m, idx

    accm = jnp.full((16,), -jnp.inf, jnp.float32)
    acci = jnp.zeros((16,), jnp.int32)
    for r in range(8):
        accm = jnp.where(lane == r, ms[r], accm)
        acci = jnp.where(lane == r, idxs[r], acci)
    resm[...] = accm
    resi[...] = acci
    pltpu.sync_copy(resm, maxs_hbm.at[wid])
    pltpu.sync_copy(resi, idxs_hbm.at[wid])


def _tc_body(x_ref, maxs_ref, idxs_ref, mstate, istate):
    i = pl.program_id(0)

    @pl.when(i == 0)
    def _():
        mstate[...] = jnp.full((B, LANE), -jnp.inf, jnp.float32)
        istate[...] = jnp.zeros((B, LANE), jnp.int32)

    base = CSC + i * TC_BLK
    lane2 = lax.broadcasted_iota(jnp.int32, (B, LANE), 1)
    m = mstate[...]
    ii = istate[...]
    for j in range(TC_BLK // LANE):
        v = x_ref[:, j * LANE:(j + 1) * LANE]
        gt = v > m
        m = jnp.where(gt, v, m)
        ii = jnp.where(gt, (base + j * LANE) + lane2, ii)
    mstate[...] = m
    istate[...] = ii

    @pl.when(i == TC_STEPS - 1)
    def _():
        bm = jnp.max(m, axis=1, keepdims=True)
        bi = jnp.min(jnp.where(m == bm, ii, jnp.int32(2**31 - 1)),
                     axis=1, keepdims=True)
        maxs_ref[...] = bm
        idxs_ref[...] = bi


_tc_argmax = pl.pallas_call(
    _tc_body,
    grid=(TC_STEPS,),
    in_specs=[pl.BlockSpec((B, TC_BLK), lambda i: (0, TC_BLK0 + i))],
    out_specs=[pl.BlockSpec((B, 1), lambda i: (0, 0)),
               pl.BlockSpec((B, 1), lambda i: (0, 0))],
    out_shape=[jax.ShapeDtypeStruct((B, 1), jnp.float32),
               jax.ShapeDtypeStruct((B, 1), jnp.int32)],
    scratch_shapes=[pltpu.VMEM((B, LANE), jnp.float32),
                    pltpu.VMEM((B, LANE), jnp.int32)],
)


def kernel(x):
    staged_m, staged_i = _sc_argmax(x)
    tc_m, tc_i = _tc_argmax(x)
    m4 = staged_m[:, :8].reshape(2, 4, 4, 8).transpose(0, 1, 3, 2).reshape(B, 4)
    i4 = staged_i[:, :8].reshape(2, 4, 4, 8).transpose(0, 1, 3, 2).reshape(B, 4)
    m5 = jnp.concatenate([m4, tc_m], axis=1)
    i5 = jnp.concatenate([i4, tc_i], axis=1)
    rowmax = m5.max(axis=1, keepdims=True)
    cand = jnp.where(m5 == rowmax, i5, jnp.int32(2**31 - 1))
    bests = cand.min(axis=1).reshape(B, 1)

    k1 = jax.random.key(1)
    k2 = jax.random.key(2)
    sampled = jax.random.randint(k1, (B,), 0, V, dtype=jnp.int32)
    bern = jax.random.bernoulli(k2, 1.0 - EPSILON, (B, 1)).astype(jnp.int32)
    return bests * bern + (1 - bern) * sampled


# speedup vs baseline: 48.8053x; 1.1324x over previous
import functools

import jax
import jax.numpy as jnp
from jax import lax
from jax.experimental import pallas as pl
from jax.experimental.pallas import tpu as pltpu
from jax.experimental.pallas import tpu_sc as plsc

B = 64
V = 1_000_000
EPSILON = 0.05

NW = 32
LANE = 128
FULL_TILES = V // LANE
TAIL = V - FULL_TILES * LANE
CT = 31
CHUNKC = CT * LANE
NCH = 39
QT = NCH * CT
QCOLS = QT * LANE
CSC = 4 * QCOLS
JGROUPS = CHUNKC // 64

TC_BLK = CHUNKC
TC_BLK0 = CSC // TC_BLK
TC_STEPS = (FULL_TILES * LANE - CSC) // TC_BLK

_mesh = plsc.VectorSubcoreMesh(core_axis_name="c", subcore_axis_name="s")

_GATHER_DNUMS = lax.GatherDimensionNumbers(
    offset_dims=(), collapsed_slice_dims=(0,), start_index_map=(0,))


def _perm16(x, pidx):
    return lax.gather(x, pidx[:, None], _GATHER_DNUMS, (1,),
                      mode=lax.GatherScatterMode.PROMISE_IN_BOUNDS)


@functools.partial(
    pl.kernel,
    mesh=_mesh,
    out_type=[
        jax.ShapeDtypeStruct((4, B), jnp.float32),
        jax.ShapeDtypeStruct((4, B), jnp.int32),
    ],
    scratch_types=[
        pltpu.VMEM((8, CHUNKC), jnp.float32),
        pltpu.VMEM((8, CHUNKC), jnp.float32),
        pltpu.VMEM((8, CHUNKC), jnp.float32),
        pltpu.VMEM((8, TAIL), jnp.float32),
        pltpu.VMEM((16,), jnp.float32),
        pltpu.VMEM((16,), jnp.int32),
        pltpu.SemaphoreType.DMA,
        pltpu.SemaphoreType.DMA,
        pltpu.SemaphoreType.DMA,
    ],
)
def _sc_argmax(x_hbm, maxs_hbm, idxs_hbm, buf0, buf1, buf2, tailbuf,
               resm, resi, sem0, sem1, sem2):
    c = lax.axis_index("c")
    s = lax.axis_index("s")
    wid = c * 16 + s
    rb = c * 4 + s // 4
    kq = s % 4
    row0 = rb * 8
    lane = lax.iota(jnp.int32, 16)

    def start_dma(t, b, sem):
        col0 = kq * QCOLS + t * CHUNKC
        pltpu.make_async_copy(
            x_hbm.at[pl.ds(row0, 8), pl.ds(col0, CHUNKC)], b, sem).start()

    def wait_dma(b, sem):
        pltpu.make_async_copy(
            x_hbm.at[pl.ds(row0, 8), pl.ds(0, CHUNKC)], b, sem).wait()

    def process(b, t, ms, idxs):
        col0 = kq * QCOLS + t * CHUNKC
        pos0 = col0 + lane

        def jbody(j, carry):
            ms, idxs, pos = carry
            ms = list(ms)
            idxs = list(idxs)
            base = j * 64
            for du in range(4):
                p = pos + (du * 16) if du else pos
                for r in range(8):
                    v = b[r, pl.ds(base + du * 16, 16)]
                    gt = v > ms[r]
                    ms[r] = jnp.where(gt, v, ms[r])
                    idxs[r] = jnp.where(gt, p, idxs[r])
            return tuple(ms), tuple(idxs), pos + 64

        ms, idxs, _ = lax.fori_loop(0, JGROUPS, jbody, (ms, idxs, pos0))
        return ms, idxs

    ms = tuple(jnp.full((16,), -jnp.inf, jnp.float32) for _ in range(8))
    idxs = tuple(jnp.zeros((16,), jnp.int32) for _ in range(8))

    bufs = (buf0, buf1, buf2)
    sems = (sem0, sem1, sem2)
    start_dma(0, buf0, sem0)
    start_dma(1, buf1, sem1)

    def ubody(u, carry):
        ms, idxs = carry
        t0 = u * 3
        for q in range(3):
            nxt = t0 + 2 + q

            @pl.when(nxt < NCH)
            def _(nxt=nxt, q=q):
                start_dma(nxt, bufs[(2 + q) % 3], sems[(2 + q) % 3])

            wait_dma(bufs[q], sems[q])
            ms, idxs = process(bufs[q], t0 + q, ms, idxs)
        return ms, idxs

    ms, idxs = lax.fori_loop(0, NCH // 3, ubody, (ms, idxs))

    pltpu.sync_copy(x_hbm.at[pl.ds(row0, 8), pl.ds(FULL_TILES * LANE, TAIL)],
                    tailbuf)
    penalty = jnp.where(kq == 3, jnp.float32(0), jnp.float32(-jnp.inf))
    ms = list(ms)
    idxs = list(idxs)
    for j in range(TAIL // 16):
        p = FULL_TILES * LANE + j * 16 + lane
        for r in range(8):
            v = tailbuf[r, pl.ds(j * 16, 16)] + penalty
            gt = v > ms[r]
            ms[r] = jnp.where(gt, v, ms[r])
            idxs[r] = jnp.where(gt, p, idxs[r])

    for r in range(8):
        m, idx = ms[r], idxs[r]
        for d in (1, 2, 4, 8):
            pidx = lane ^ d
            mp = _perm16(m, pidx)
            ip = _perm16(idx, pidx)
            better = (mp > m) | ((mp == m) & (ip < idx))
            m = jnp.where(better, mp, m)
            idx = jnp.where(better, ip, idx)
        ms[r], idxs[r] = m, idx

    accm = jnp.full((16,), -jnp.inf, jnp.float32)
    acci = jnp.zeros((16,), jnp.int32)
    for r in range(8):
        accm = jnp.where(lane == r, ms[r], accm)
        acci = jnp.where(lane == r, idxs[r], acci)
    resm[...] = accm
    resi[...] = acci
    pltpu.sync_copy(resm.at[pl.ds(0, 8)], maxs_hbm.at[kq, pl.ds(row0, 8)])
    pltpu.sync_copy(resi.at[pl.ds(0, 8)], idxs_hbm.at[kq, pl.ds(row0, 8)])


def _tc_body(x_ref, maxs_ref, idxs_ref, mstate, istate):
    i = pl.program_id(0)

    @pl.when(i == 0)
    def _():
        mstate[...] = jnp.full((B, LANE), -jnp.inf, jnp.float32)
        istate[...] = jnp.zeros((B, LANE), jnp.int32)

    base = CSC + i * TC_BLK
    lane2 = lax.broadcasted_iota(jnp.int32, (B, LANE), 1)
    m = mstate[...]
    ii = istate[...]
    for j in range(TC_BLK // LANE):
        v = x_ref[:, j * LANE:(j + 1) * LANE]
        gt = v > m
        m = jnp.where(gt, v, m)
        ii = jnp.where(gt, (base + j * LANE) + lane2, ii)
    mstate[...] = m
    istate[...] = ii

    @pl.when(i == TC_STEPS - 1)
    def _():
        bm = jnp.max(m, axis=1, keepdims=True)
        bi = jnp.min(jnp.where(m == bm, ii, jnp.int32(2**31 - 1)),
                     axis=1, keepdims=True)
        maxs_ref[...] = bm
        idxs_ref[...] = bi


_tc_argmax = pl.pallas_call(
    _tc_body,
    grid=(TC_STEPS,),
    in_specs=[pl.BlockSpec((B, TC_BLK), lambda i: (0, TC_BLK0 + i))],
    out_specs=[pl.BlockSpec((B, 1), lambda i: (0, 0)),
               pl.BlockSpec((B, 1), lambda i: (0, 0))],
    out_shape=[jax.ShapeDtypeStruct((B, 1), jnp.float32),
               jax.ShapeDtypeStruct((B, 1), jnp.int32)],
    scratch_shapes=[pltpu.VMEM((B, LANE), jnp.float32),
                    pltpu.VMEM((B, LANE), jnp.int32)],
)


def kernel(x):
    staged_m, staged_i = _sc_argmax(x)
    tc_m, tc_i = _tc_argmax(x)
    m5 = jnp.concatenate([staged_m, tc_m.reshape(1, B)], axis=0)
    i5 = jnp.concatenate([staged_i, tc_i.reshape(1, B)], axis=0)
    rowmax = m5.max(axis=0, keepdims=True)
    cand = jnp.where(m5 == rowmax, i5, jnp.int32(2**31 - 1))
    bests = cand.min(axis=0).reshape(B, 1)

    k1 = jax.random.key(1)
    k2 = jax.random.key(2)
    sampled = jax.random.randint(k1, (B,), 0, V, dtype=jnp.int32)
    bern = jax.random.bernoulli(k2, 1.0 - EPSILON, (B, 1)).astype(jnp.int32)
    return bests * bern + (1 - bern) * sampled
